# baseline (speedup 1.0000x reference)
import jax
import jax.numpy as jnp
from jax.experimental import pallas as pl
from jax.experimental.pallas import tpu as pltpu

_CH = 2000
_DEPTH = 4


def _stream_mlp_kernel(prev_hbm, expr_hbm, mask_ref, w_ref, b_ref, out_hbm,
                       pbuf, ebuf, obuf, in_sems, out_sems):
    n = prev_hbm.shape[0]
    node_dim = prev_hbm.shape[1]
    nch = n // _CH
    w = w_ref[...]
    wa = w[:, :node_dim].astype(jnp.bfloat16)
    wb = w[:, node_dim:].astype(jnp.bfloat16)
    dn = (((1,), (1,)), ((), ()))
    bias = b_ref[...]

    _H = _CH // 2

    def in_copies(i):
        slot = i % _DEPTH
        lo = pl.ds(i * _CH, _H)
        hi = pl.ds(i * _CH + _H, _H)
        return (
            pltpu.make_async_copy(prev_hbm.at[lo, :], pbuf.at[slot, :_H],
                                  in_sems.at[slot, 0]),
            pltpu.make_async_copy(prev_hbm.at[hi, :], pbuf.at[slot, _H:],
                                  in_sems.at[slot, 1]),
            pltpu.make_async_copy(expr_hbm.at[lo, :], ebuf.at[slot, :_H],
                                  in_sems.at[slot, 2]),
            pltpu.make_async_copy(expr_hbm.at[hi, :], ebuf.at[slot, _H:],
                                  in_sems.at[slot, 3]),
        )

    def out_copy(i):
        slot = i % _DEPTH
        rows = pl.ds(i * _CH, _CH)
        return pltpu.make_async_copy(obuf.at[slot], out_hbm.at[rows, :],
                                     out_sems.at[slot])

    for i in range(min(_DEPTH, nch)):
        for cp in in_copies(i):
            cp.start()

    for i in range(nch):
        slot = i % _DEPTH
        keep = mask_ref[pl.ds(i * _CH, _CH)].reshape(_CH, 1) > 0.5
        for cp in in_copies(i):
            cp.wait()
        prev = pbuf[slot]
        h = jax.lax.dot_general(prev.astype(jnp.bfloat16), wa, dn,
                                preferred_element_type=jnp.float32)
        h = h + jax.lax.dot_general(ebuf[slot].astype(jnp.bfloat16), wb, dn,
                                    preferred_element_type=jnp.float32)
        h = jnp.maximum(h + bias, 0.0)
        if i >= _DEPTH:
            out_copy(i - _DEPTH).wait()
        obuf[slot] = jnp.where(keep, h, prev)
        out_copy(i).start()
        if i + _DEPTH < nch:
            for cp in in_copies(i + _DEPTH):
                cp.start()

    for i in range(max(nch - _DEPTH, 0), nch):
        out_copy(i).wait()


def kernel(previous_cfg_nodes_encodings, cfg_combined_expressions_encodings,
           cfg_nodes_has_expression_mask, W, b):
    n, node_dim = previous_cfg_nodes_encodings.shape
    in_dim = W.shape[1]
    b_row = b.reshape(1, node_dim)
    mask_f = cfg_nodes_has_expression_mask.astype(jnp.float32)
    return pl.pallas_call(
        _stream_mlp_kernel,
        in_specs=[
            pl.BlockSpec(memory_space=pltpu.MemorySpace.HBM),
            pl.BlockSpec(memory_space=pltpu.MemorySpace.HBM),
            pl.BlockSpec(memory_space=pltpu.MemorySpace.VMEM),
            pl.BlockSpec(memory_space=pltpu.MemorySpace.VMEM),
            pl.BlockSpec(memory_space=pltpu.MemorySpace.VMEM),
        ],
        out_specs=pl.BlockSpec(memory_space=pltpu.MemorySpace.HBM),
        out_shape=jax.ShapeDtypeStruct((n, node_dim), jnp.float32),
        scratch_shapes=[
            pltpu.VMEM((_DEPTH, _CH, node_dim), jnp.float32),
            pltpu.VMEM((_DEPTH, _CH, node_dim), jnp.float32),
            pltpu.VMEM((_DEPTH, _CH, node_dim), jnp.float32),
            pltpu.SemaphoreType.DMA((_DEPTH, 4)),
            pltpu.SemaphoreType.DMA((_DEPTH,)),
        ],
    )(previous_cfg_nodes_encodings, cfg_combined_expressions_encodings,
      mask_f, W, b_row)


# speedup vs baseline: 1.0109x; 1.0109x over previous
import jax
import jax.numpy as jnp
from jax.experimental import pallas as pl
from jax.experimental.pallas import tpu as pltpu

_CH = 2000
_DEPTH = 4


def _stream_mlp_kernel(prev_hbm, expr_hbm, mask_ref, w_ref, b_ref, out_hbm,
                       pbuf, ebuf, obuf, in_sems, out_sems):
    n = prev_hbm.shape[0]
    node_dim = prev_hbm.shape[1]
    nch = n // _CH
    w = w_ref[...]
    wa = w[:, :node_dim].astype(jnp.bfloat16)
    wb = w[:, node_dim:].astype(jnp.bfloat16)
    dn = (((1,), (1,)), ((), ()))
    bias = b_ref[...]

    def in_copies(i):
        slot = i % _DEPTH
        rows = pl.ds(i * _CH, _CH)
        return (
            pltpu.make_async_copy(prev_hbm.at[rows, :], pbuf.at[slot],
                                  in_sems.at[slot, 0]),
            pltpu.make_async_copy(expr_hbm.at[rows, :], ebuf.at[slot],
                                  in_sems.at[slot, 1]),
        )

    def out_copy(i):
        slot = i % _DEPTH
        rows = pl.ds(i * _CH, _CH)
        return pltpu.make_async_copy(obuf.at[slot], out_hbm.at[rows, :],
                                     out_sems.at[slot])

    for i in range(min(_DEPTH, nch)):
        for cp in in_copies(i):
            cp.start()

    for i in range(nch):
        slot = i % _DEPTH
        keep = mask_ref[pl.ds(i * _CH, _CH)].reshape(_CH, 1) > 0.5
        for cp in in_copies(i):
            cp.wait()
        prev = pbuf[slot]
        h = jax.lax.dot_general(prev.astype(jnp.bfloat16), wa, dn,
                                preferred_element_type=jnp.float32)
        h = h + jax.lax.dot_general(ebuf[slot].astype(jnp.bfloat16), wb, dn,
                                    preferred_element_type=jnp.float32)
        h = jnp.maximum(h + bias, 0.0)
        if i >= _DEPTH:
            out_copy(i - _DEPTH).wait()
        obuf[slot] = jnp.where(keep, h, prev)
        out_copy(i).start()
        if i + _DEPTH < nch:
            for cp in in_copies(i + _DEPTH):
                cp.start()

    for i in range(max(nch - _DEPTH, 0), nch):
        out_copy(i).wait()


def kernel(previous_cfg_nodes_encodings, cfg_combined_expressions_encodings,
           cfg_nodes_has_expression_mask, W, b):
    n, node_dim = previous_cfg_nodes_encodings.shape
    in_dim = W.shape[1]
    b_row = b.reshape(1, node_dim)
    mask_f = cfg_nodes_has_expression_mask.astype(jnp.float32)
    return pl.pallas_call(
        _stream_mlp_kernel,
        in_specs=[
            pl.BlockSpec(memory_space=pltpu.MemorySpace.HBM),
            pl.BlockSpec(memory_space=pltpu.MemorySpace.HBM),
            pl.BlockSpec(memory_space=pltpu.MemorySpace.VMEM),
            pl.BlockSpec(memory_space=pltpu.MemorySpace.VMEM),
            pl.BlockSpec(memory_space=pltpu.MemorySpace.VMEM),
        ],
        out_specs=pl.BlockSpec(memory_space=pltpu.MemorySpace.HBM),
        out_shape=jax.ShapeDtypeStruct((n, node_dim), jnp.float32),
        scratch_shapes=[
            pltpu.VMEM((_DEPTH, _CH, node_dim), jnp.float32),
            pltpu.VMEM((_DEPTH, _CH, node_dim), jnp.float32),
            pltpu.VMEM((_DEPTH, _CH, node_dim), jnp.float32),
            pltpu.SemaphoreType.DMA((_DEPTH, 2)),
            pltpu.SemaphoreType.DMA((_DEPTH,)),
        ],
    )(previous_cfg_nodes_encodings, cfg_combined_expressions_encodings,
      mask_f, W, b_row)
